# Initial kernel scaffold; baseline (speedup 1.0000x reference)
import jax
import jax.numpy as jnp
from jax.experimental import pallas as pl


def kernel(x, edge_index, edge_weight, W_neigh0, W_self0, b_neigh0, W_neigh_h, W_self_h, b_neigh_h, W_fc1, b_fc1, W_out, b_out):
    raise NotImplementedError("write your pallas kernel here")


# speedup vs baseline: 8.0933x; 8.0933x over previous
import functools

import jax
import jax.numpy as jnp
from jax import lax
from jax.experimental import pallas as pl
from jax.experimental.pallas import tpu as pltpu
from jax.experimental.pallas import tpu_sc as plsc

_NC = 2
_NS = 16
_NW = _NC * _NS
_B = 80


@functools.lru_cache(maxsize=None)
def _edge_agg(n: int, e: int, h: int):
    epw = e // _NW
    nblk = epw // _B
    assert nblk * _B == epw
    rchunk = 80
    nchunks = n // rchunk
    assert nchunks * rchunk == n
    chunks_per_tile = -(-nchunks // _NS)

    mesh = plsc.VectorSubcoreMesh(core_axis_name="c", subcore_axis_name="s")

    @functools.partial(
        pl.kernel,
        out_type=jax.ShapeDtypeStruct((_NC, n, h), jnp.float32),
        mesh=mesh,
        compiler_params=pltpu.CompilerParams(
            needs_layout_passes=False, use_tc_tiling_on_sc=False),
        scratch_types=[
            pltpu.VMEM((epw,), jnp.int32),
            pltpu.VMEM((nblk, _B), jnp.int32),
            pltpu.VMEM((epw,), jnp.float32),
            pltpu.VMEM((_B, h), jnp.float32),
            pltpu.VMEM((rchunk, h), jnp.float32),
            pltpu.VMEM_SHARED((n, h), jnp.float32),
            pltpu.SemaphoreType.DMA,
        ],
    )
    def agg_kernel(h_hbm, src_hbm, dst_hbm, ew_hbm, out_hbm,
                   src_v, dst_v, ew_v, rows_v, zero_v, agg_sh, sem):
        cid = lax.axis_index("c")
        sid = lax.axis_index("s")
        wid = cid * _NS + sid

        z16 = jnp.zeros((16,), jnp.float32)

        def zero_body(i, _):
            zero_v[i, pl.ds(0, 16)] = z16
            zero_v[i, pl.ds(16, 16)] = z16
            return 0

        lax.fori_loop(0, rchunk, zero_body, 0)
        for k in range(chunks_per_tile):
            c = sid + _NS * k

            @pl.when(c < nchunks)
            def _():
                pltpu.sync_copy(zero_v, agg_sh.at[pl.ds(c * rchunk, rchunk)])

        pltpu.sync_copy(src_hbm.at[wid], src_v)
        pltpu.sync_copy(dst_hbm.at[wid], dst_v)
        pltpu.sync_copy(ew_hbm.at[wid], ew_v)
        plsc.subcore_barrier()

        def blk_body(j, _):
            idx = src_v.at[pl.ds(j * _B, _B)]
            pltpu.async_copy(h_hbm.at[idx], rows_v, sem).wait()

            def scale_body(i, _):
                wsplat = plsc.load_gather(
                    ew_v, [jnp.full((16,), j * _B + i, jnp.int32)])
                rows_v[i, pl.ds(0, 16)] = rows_v[i, pl.ds(0, 16)] * wsplat
                rows_v[i, pl.ds(16, 16)] = rows_v[i, pl.ds(16, 16)] * wsplat
                return 0

            lax.fori_loop(0, _B, scale_body, 0)
            pltpu.sync_copy(rows_v, agg_sh.at[dst_v.at[j]], add=True)
            return 0

        lax.fori_loop(0, nblk, blk_body, 0)
        plsc.subcore_barrier()
        for k in range(chunks_per_tile):
            c = sid + _NS * k

            @pl.when(c < nchunks)
            def _():
                pltpu.sync_copy(agg_sh.at[pl.ds(c * rchunk, rchunk)],
                                out_hbm.at[cid, pl.ds(c * rchunk, rchunk)])

    return agg_kernel


def _tc_call(body, out_shapes, *args):
    return pl.pallas_call(
        body,
        out_shape=[jax.ShapeDtypeStruct(s, jnp.float32) for s in out_shapes],
    )(*args)


def _mm2_body(x_ref, wn_ref, ws_ref, hm_ref, sm_ref):
    x = x_ref[...]
    hm_ref[...] = jnp.dot(x, wn_ref[...], preferred_element_type=jnp.float32)
    sm_ref[...] = jnp.dot(x, ws_ref[...], preferred_element_type=jnp.float32)


def _combine_body(agg_ref, s_ref, b_ref, wn_ref, ws_ref, hm_ref, sm_ref):
    hcur = jax.nn.relu(agg_ref[0] + agg_ref[1] + b_ref[...] + s_ref[...])
    hm_ref[...] = jnp.dot(hcur, wn_ref[...], preferred_element_type=jnp.float32)
    sm_ref[...] = jnp.dot(hcur, ws_ref[...], preferred_element_type=jnp.float32)


def _final_body(agg_ref, s_ref, b_ref, wfc1_ref, bfc1_ref, wout_ref, bout_ref,
                out_ref):
    hcur = jax.nn.relu(agg_ref[0] + agg_ref[1] + b_ref[...] + s_ref[...])
    hg = jnp.sum(hcur, axis=0, keepdims=True)
    hg2 = jax.nn.relu(
        jnp.dot(hg, wfc1_ref[...], preferred_element_type=jnp.float32)
        + bfc1_ref[...])
    o = jax.nn.relu(
        jnp.dot(hg2, wout_ref[...], preferred_element_type=jnp.float32)
        + bout_ref[...])
    out_ref[...] = jax.nn.softmax(o, axis=1)


def kernel(x, edge_index, edge_weight, W_neigh0, W_self0, b_neigh0,
           W_neigh_h, W_self_h, b_neigh_h, W_fc1, b_fc1, W_out, b_out):
    n, d = x.shape
    e = edge_index.shape[1]
    h = W_neigh0.shape[1]
    epw = e // _NW

    src_r = edge_index[0].reshape(_NW, epw)
    dst_r = edge_index[1].reshape(_NW, epw // _B, _B)
    ew_r = edge_weight.reshape(_NW, epw)

    agg_fn = _edge_agg(n, e, h)

    hm, sm = _tc_call(_mm2_body, [(n, h), (n, h)], x, W_neigh0, W_self0)

    biases = [b_neigh0.reshape(1, h)] + [b_neigh_h[i].reshape(1, h)
                                         for i in range(3)]
    for i in range(3):
        agg = agg_fn(hm, src_r, dst_r, ew_r)
        hm, sm = _tc_call(_combine_body, [(n, h), (n, h)],
                          agg, sm, biases[i], W_neigh_h[i], W_self_h[i])

    agg = agg_fn(hm, src_r, dst_r, ew_r)
    (out,) = _tc_call(_final_body, [(1, 4)],
                      agg, sm, biases[3], W_fc1, b_fc1.reshape(1, 8),
                      W_out, b_out.reshape(1, 4))
    return out


# speedup vs baseline: 12.6058x; 1.5576x over previous
import functools

import jax
import jax.numpy as jnp
from jax import lax
from jax.experimental import pallas as pl
from jax.experimental.pallas import tpu as pltpu
from jax.experimental.pallas import tpu_sc as plsc

_NC = 2
_NS = 16
_NW = _NC * _NS
_B = 80


@functools.lru_cache(maxsize=None)
def _edge_agg(n: int, e: int, h: int):
    epw = e // _NW
    nblk = epw // _B
    assert nblk * _B == epw
    rchunk = 80
    nchunks = n // rchunk
    assert nchunks * rchunk == n
    chunks_per_tile = -(-nchunks // _NS)

    mesh = plsc.VectorSubcoreMesh(core_axis_name="c", subcore_axis_name="s")

    @functools.partial(
        pl.kernel,
        out_type=jax.ShapeDtypeStruct((_NC, n, h), jnp.float32),
        mesh=mesh,
        compiler_params=pltpu.CompilerParams(
            needs_layout_passes=False, use_tc_tiling_on_sc=False),
        scratch_types=[
            pltpu.VMEM((epw,), jnp.int32),
            pltpu.VMEM((nblk, _B), jnp.int32),
            pltpu.VMEM((epw,), jnp.float32),
            pltpu.VMEM((_B, h), jnp.float32),
            pltpu.VMEM((_B, h), jnp.float32),
            pltpu.VMEM((rchunk, h), jnp.float32),
            pltpu.VMEM_SHARED((n, h), jnp.float32),
            pltpu.SemaphoreType.DMA,
            pltpu.SemaphoreType.DMA,
        ],
    )
    def agg_kernel(h_hbm, src_hbm, dst_hbm, ew_hbm, out_hbm,
                   src_v, dst_v, ew_v, rows0, rows1, zero_v, agg_sh,
                   sem0, sem1):
        cid = lax.axis_index("c")
        sid = lax.axis_index("s")
        wid = cid * _NS + sid

        pltpu.sync_copy(src_hbm.at[wid], src_v)
        pltpu.sync_copy(dst_hbm.at[wid], dst_v)
        pltpu.sync_copy(ew_hbm.at[wid], ew_v)

        z16 = jnp.zeros((16,), jnp.float32)

        def zero_body(i, _):
            zero_v[i, pl.ds(0, 16)] = z16
            zero_v[i, pl.ds(16, 16)] = z16
            return 0

        lax.fori_loop(0, rchunk, zero_body, 0)
        for k in range(chunks_per_tile):
            c = sid + _NS * k

            @pl.when(c < nchunks)
            def _():
                pltpu.sync_copy(zero_v, agg_sh.at[pl.ds(c * rchunk, rchunk)])

        plsc.subcore_barrier()

        pltpu.async_copy(h_hbm.at[src_v.at[pl.ds(0, _B)]], rows0, sem0)

        def blk_body(j, _):
            def phase(cur, csem, nxt, nsem):
                pltpu.make_async_copy(
                    h_hbm.at[src_v.at[pl.ds(j * _B, _B)]], cur, csem).wait()

                @pl.when(j + 1 < nblk)
                def _():
                    idx = src_v.at[pl.ds((j + 1) * _B, _B)]
                    pltpu.async_copy(h_hbm.at[idx], nxt, nsem)

                def scale_body(i, _):
                    wsplat = plsc.load_gather(
                        ew_v, [jnp.full((16,), j * _B + i, jnp.int32)])
                    cur[i, pl.ds(0, 16)] = cur[i, pl.ds(0, 16)] * wsplat
                    cur[i, pl.ds(16, 16)] = cur[i, pl.ds(16, 16)] * wsplat
                    return 0

                lax.fori_loop(0, _B, scale_body, 0, unroll=8)
                pltpu.sync_copy(cur, agg_sh.at[dst_v.at[j]], add=True)

            @pl.when(j % 2 == 0)
            def _():
                phase(rows0, sem0, rows1, sem1)

            @pl.when(j % 2 == 1)
            def _():
                phase(rows1, sem1, rows0, sem0)

            return 0

        lax.fori_loop(0, nblk, blk_body, 0)
        plsc.subcore_barrier()
        for k in range(chunks_per_tile):
            c = sid + _NS * k

            @pl.when(c < nchunks)
            def _():
                pltpu.sync_copy(agg_sh.at[pl.ds(c * rchunk, rchunk)],
                                out_hbm.at[cid, pl.ds(c * rchunk, rchunk)])

    return agg_kernel


def _tc_call(body, out_shapes, *args):
    return pl.pallas_call(
        body,
        out_shape=[jax.ShapeDtypeStruct(s, jnp.float32) for s in out_shapes],
    )(*args)


def _mm2_body(x_ref, wn_ref, ws_ref, hm_ref, sm_ref):
    x = x_ref[...]
    hm_ref[...] = jnp.dot(x, wn_ref[...], preferred_element_type=jnp.float32)
    sm_ref[...] = jnp.dot(x, ws_ref[...], preferred_element_type=jnp.float32)


def _combine_body(agg_ref, s_ref, b_ref, wn_ref, ws_ref, hm_ref, sm_ref):
    hcur = jax.nn.relu(agg_ref[0] + agg_ref[1] + b_ref[...] + s_ref[...])
    hm_ref[...] = jnp.dot(hcur, wn_ref[...], preferred_element_type=jnp.float32)
    sm_ref[...] = jnp.dot(hcur, ws_ref[...], preferred_element_type=jnp.float32)


def _final_body(agg_ref, s_ref, b_ref, wfc1_ref, bfc1_ref, wout_ref, bout_ref,
                out_ref):
    hcur = jax.nn.relu(agg_ref[0] + agg_ref[1] + b_ref[...] + s_ref[...])
    hg = jnp.sum(hcur, axis=0, keepdims=True)
    hg2 = jax.nn.relu(
        jnp.dot(hg, wfc1_ref[...], preferred_element_type=jnp.float32)
        + bfc1_ref[...])
    o = jax.nn.relu(
        jnp.dot(hg2, wout_ref[...], preferred_element_type=jnp.float32)
        + bout_ref[...])
    out_ref[...] = jax.nn.softmax(o, axis=1)


def kernel(x, edge_index, edge_weight, W_neigh0, W_self0, b_neigh0,
           W_neigh_h, W_self_h, b_neigh_h, W_fc1, b_fc1, W_out, b_out):
    n, d = x.shape
    e = edge_index.shape[1]
    h = W_neigh0.shape[1]
    epw = e // _NW

    src_r = edge_index[0].reshape(_NW, epw)
    dst_r = edge_index[1].reshape(_NW, epw // _B, _B)
    ew_r = edge_weight.reshape(_NW, epw)

    agg_fn = _edge_agg(n, e, h)

    hm, sm = _tc_call(_mm2_body, [(n, h), (n, h)], x, W_neigh0, W_self0)

    biases = [b_neigh0.reshape(1, h)] + [b_neigh_h[i].reshape(1, h)
                                         for i in range(3)]
    for i in range(3):
        agg = agg_fn(hm, src_r, dst_r, ew_r)
        hm, sm = _tc_call(_combine_body, [(n, h), (n, h)],
                          agg, sm, biases[i], W_neigh_h[i], W_self_h[i])

    agg = agg_fn(hm, src_r, dst_r, ew_r)
    (out,) = _tc_call(_final_body, [(1, 4)],
                      agg, sm, biases[3], W_fc1, b_fc1.reshape(1, 8),
                      W_out, b_out.reshape(1, 4))
    return out


# speedup vs baseline: 14.0693x; 1.1161x over previous
import functools

import jax
import jax.numpy as jnp
from jax import lax
from jax.experimental import pallas as pl
from jax.experimental.pallas import tpu as pltpu
from jax.experimental.pallas import tpu_sc as plsc

_NC = 2
_NS = 16
_NW = _NC * _NS
_B = 80


@functools.lru_cache(maxsize=None)
def _edge_agg(n: int, e: int, h: int):
    epw = e // _NW
    nblk = epw // _B
    assert nblk * _B == epw
    rchunk = 80
    nchunks = n // rchunk
    assert nchunks * rchunk == n
    chunks_per_tile = -(-nchunks // _NS)

    mesh = plsc.VectorSubcoreMesh(core_axis_name="c", subcore_axis_name="s")

    @functools.partial(
        pl.kernel,
        out_type=jax.ShapeDtypeStruct((_NC, n, h), jnp.float32),
        mesh=mesh,
        compiler_params=pltpu.CompilerParams(
            needs_layout_passes=False, use_tc_tiling_on_sc=False),
        scratch_types=[
            pltpu.VMEM((epw,), jnp.int32),
            pltpu.VMEM((nblk, _B), jnp.int32),
            pltpu.VMEM((epw,), jnp.float32),
            pltpu.VMEM((_B, h), jnp.float32),
            pltpu.VMEM((_B, h), jnp.float32),
            pltpu.VMEM((_B, h), jnp.float32),
            pltpu.VMEM((_B, h), jnp.float32),
            pltpu.VMEM((rchunk, h), jnp.float32),
            pltpu.VMEM_SHARED((n, h), jnp.float32),
            [pltpu.SemaphoreType.DMA] * 4,
            [pltpu.SemaphoreType.DMA] * 4,
        ],
    )
    def agg_kernel(h_hbm, src_hbm, dst_hbm, ew_hbm, out_hbm,
                   src_v, dst_v, ew_v, rows0, rows1, rows2, rows3,
                   zero_v, agg_sh, gsems, ssems):
        cid = lax.axis_index("c")
        sid = lax.axis_index("s")
        wid = cid * _NS + sid

        pltpu.sync_copy(src_hbm.at[wid], src_v)
        pltpu.sync_copy(dst_hbm.at[wid], dst_v)
        pltpu.sync_copy(ew_hbm.at[wid], ew_v)

        z16 = jnp.zeros((16,), jnp.float32)

        def zero_body(i, _):
            zero_v[i, pl.ds(0, 16)] = z16
            zero_v[i, pl.ds(16, 16)] = z16
            return 0

        lax.fori_loop(0, rchunk, zero_body, 0)
        for k in range(chunks_per_tile):
            c = sid + _NS * k

            @pl.when(c < nchunks)
            def _():
                pltpu.sync_copy(zero_v, agg_sh.at[pl.ds(c * rchunk, rchunk)])

        plsc.subcore_barrier()

        bufs = (rows0, rows1, rows2, rows3)

        def gather_desc(jj, p):
            return pltpu.make_async_copy(
                h_hbm.at[src_v.at[pl.ds(jj * _B, _B)]], bufs[p], gsems[p])

        def scatter_desc(jj, p):
            return pltpu.make_async_copy(
                bufs[p], agg_sh.at[dst_v.at[jj]], ssems[p])

        pltpu.async_copy(h_hbm.at[src_v.at[pl.ds(0, _B)]], rows0, gsems[0])
        pltpu.async_copy(h_hbm.at[src_v.at[pl.ds(_B, _B)]], rows1, gsems[1])

        def blk_body(j, _):
            def phase(p):
                r = (p + 2) % 4
                gather_desc(j, p).wait()

                cur = bufs[p]

                def scale_body(i, _):
                    wsplat = plsc.load_gather(
                        ew_v, [jnp.full((16,), j * _B + i, jnp.int32)])
                    cur[i, pl.ds(0, 16)] = cur[i, pl.ds(0, 16)] * wsplat
                    cur[i, pl.ds(16, 16)] = cur[i, pl.ds(16, 16)] * wsplat
                    return 0

                lax.fori_loop(0, _B, scale_body, 0, unroll=8)
                pltpu.async_copy(cur, agg_sh.at[dst_v.at[j]], ssems[p],
                                 add=True)

                @pl.when(j + 2 < nblk)
                def _():
                    @pl.when(j >= 2)
                    def _():
                        scatter_desc(j - 2, r).wait()

                    idx = src_v.at[pl.ds((j + 2) * _B, _B)]
                    pltpu.async_copy(h_hbm.at[idx], bufs[r], gsems[r])

            for p in range(4):
                @pl.when(j % 4 == p)
                def _(p=p):
                    phase(p)

            return 0

        lax.fori_loop(0, nblk, blk_body, 0)
        scatter_desc(nblk - 2, (nblk - 2) % 4).wait()
        scatter_desc(nblk - 1, (nblk - 1) % 4).wait()
        plsc.subcore_barrier()
        for k in range(chunks_per_tile):
            c = sid + _NS * k

            @pl.when(c < nchunks)
            def _():
                pltpu.sync_copy(agg_sh.at[pl.ds(c * rchunk, rchunk)],
                                out_hbm.at[cid, pl.ds(c * rchunk, rchunk)])

    return agg_kernel


def _tc_call(body, out_shapes, *args):
    return pl.pallas_call(
        body,
        out_shape=[jax.ShapeDtypeStruct(s, jnp.float32) for s in out_shapes],
    )(*args)


def _mm2_body(x_ref, wn_ref, ws_ref, hm_ref, sm_ref):
    x = x_ref[...]
    hm_ref[...] = jnp.dot(x, wn_ref[...], preferred_element_type=jnp.float32)
    sm_ref[...] = jnp.dot(x, ws_ref[...], preferred_element_type=jnp.float32)


def _combine_body(agg_ref, s_ref, b_ref, wn_ref, ws_ref, hm_ref, sm_ref):
    hcur = jax.nn.relu(agg_ref[0] + agg_ref[1] + b_ref[...] + s_ref[...])
    hm_ref[...] = jnp.dot(hcur, wn_ref[...], preferred_element_type=jnp.float32)
    sm_ref[...] = jnp.dot(hcur, ws_ref[...], preferred_element_type=jnp.float32)


def _final_body(agg_ref, s_ref, b_ref, wfc1_ref, bfc1_ref, wout_ref, bout_ref,
                out_ref):
    hcur = jax.nn.relu(agg_ref[0] + agg_ref[1] + b_ref[...] + s_ref[...])
    hg = jnp.sum(hcur, axis=0, keepdims=True)
    hg2 = jax.nn.relu(
        jnp.dot(hg, wfc1_ref[...], preferred_element_type=jnp.float32)
        + bfc1_ref[...])
    o = jax.nn.relu(
        jnp.dot(hg2, wout_ref[...], preferred_element_type=jnp.float32)
        + bout_ref[...])
    out_ref[...] = jax.nn.softmax(o, axis=1)


def kernel(x, edge_index, edge_weight, W_neigh0, W_self0, b_neigh0,
           W_neigh_h, W_self_h, b_neigh_h, W_fc1, b_fc1, W_out, b_out):
    n, d = x.shape
    e = edge_index.shape[1]
    h = W_neigh0.shape[1]
    epw = e // _NW

    src_r = edge_index[0].reshape(_NW, epw)
    dst_r = edge_index[1].reshape(_NW, epw // _B, _B)
    ew_r = edge_weight.reshape(_NW, epw)

    agg_fn = _edge_agg(n, e, h)

    hm, sm = _tc_call(_mm2_body, [(n, h), (n, h)], x, W_neigh0, W_self0)

    biases = [b_neigh0.reshape(1, h)] + [b_neigh_h[i].reshape(1, h)
                                         for i in range(3)]
    for i in range(3):
        agg = agg_fn(hm, src_r, dst_r, ew_r)
        hm, sm = _tc_call(_combine_body, [(n, h), (n, h)],
                          agg, sm, biases[i], W_neigh_h[i], W_self_h[i])

    agg = agg_fn(hm, src_r, dst_r, ew_r)
    (out,) = _tc_call(_final_body, [(1, 4)],
                      agg, sm, biases[3], W_fc1, b_fc1.reshape(1, 8),
                      W_out, b_out.reshape(1, 4))
    return out
